# baseline (device time: 7946 ns/iter reference)
import jax
import jax.numpy as jnp
from jax import lax
from jax.experimental import pallas as pl
from jax.experimental.pallas import tpu as pltpu

N_GLOBAL_COLS = 1024
EPS = 1e-5
LANES = 128


def kernel(x, gamma):
    m, n = x.shape
    nblk = m // LANES
    gamma2d = gamma.reshape(1, n)

    def body(x_ref, g_ref, out_ref, packed_ref, recv_ref, send_sem, recv_sem):
        my_x = lax.axis_index("x")
        my_y = lax.axis_index("y")
        nbr = (my_x, 1 - my_y)

        barrier_sem = pltpu.get_barrier_semaphore()
        pl.semaphore_signal(
            barrier_sem, inc=1, device_id=nbr,
            device_id_type=pl.DeviceIdType.MESH,
        )
        pl.semaphore_wait(barrier_sem, 1)

        xv = x_ref[:, :]
        ssq = jnp.sum(xv * xv, axis=1, keepdims=True)

        f32 = jnp.float32
        L = (
            lax.broadcasted_iota(jnp.int32, (m, LANES), 0) % LANES
            == lax.broadcasted_iota(jnp.int32, (m, LANES), 1)
        ).astype(f32)
        Rt = (
            lax.broadcasted_iota(jnp.int32, (nblk, m), 1) // LANES
            == lax.broadcasted_iota(jnp.int32, (nblk, m), 0)
        ).astype(f32)
        R = (
            lax.broadcasted_iota(jnp.int32, (m, nblk), 0) // LANES
            == lax.broadcasted_iota(jnp.int32, (m, nblk), 1)
        ).astype(f32)

        packed_ref[:, :] = jnp.dot(
            Rt, L * ssq, preferred_element_type=f32
        )

        rdma = pltpu.make_async_remote_copy(
            src_ref=packed_ref,
            dst_ref=recv_ref,
            send_sem=send_sem,
            recv_sem=recv_sem,
            device_id=nbr,
            device_id_type=pl.DeviceIdType.MESH,
        )
        rdma.start()
        xg = xv * g_ref[0, :]
        rdma.wait()

        total = packed_ref[:, :] + recv_ref[:, :]
        v = jnp.sum(
            jnp.dot(R, total, preferred_element_type=f32) * L,
            axis=1,
            keepdims=True,
        )
        inv_rms = lax.rsqrt(v * (1.0 / N_GLOBAL_COLS) + EPS)
        out_ref[:, :] = xg * inv_rms

    return pl.pallas_call(
        body,
        out_shape=jax.ShapeDtypeStruct((m, n), x.dtype),
        in_specs=[
            pl.BlockSpec(memory_space=pltpu.VMEM),
            pl.BlockSpec(memory_space=pltpu.VMEM),
        ],
        out_specs=pl.BlockSpec(memory_space=pltpu.VMEM),
        scratch_shapes=[
            pltpu.VMEM((nblk, LANES), jnp.float32),
            pltpu.VMEM((nblk, LANES), jnp.float32),
            pltpu.SemaphoreType.DMA,
            pltpu.SemaphoreType.DMA,
        ],
        compiler_params=pltpu.CompilerParams(collective_id=0),
    )(x, gamma2d)


# device time: 7931 ns/iter; 1.0019x vs baseline; 1.0019x over previous
import jax
import jax.numpy as jnp
from jax import lax
from jax.experimental import pallas as pl
from jax.experimental.pallas import tpu as pltpu

N_GLOBAL_COLS = 1024
EPS = 1e-5
LANES = 128


def kernel(x, gamma):
    m, n = x.shape
    nblk = m // LANES
    gamma2d = gamma.reshape(1, n)

    def body(x_ref, g_ref, out_ref, packed_ref, recv_ref, send_sem, recv_sem):
        my_x = lax.axis_index("x")
        my_y = lax.axis_index("y")
        nbr = (my_x, 1 - my_y)

        barrier_sem = pltpu.get_barrier_semaphore()
        pl.semaphore_signal(
            barrier_sem, inc=1, device_id=nbr,
            device_id_type=pl.DeviceIdType.MESH,
        )

        xv = x_ref[:, :]
        ssq = jnp.sum(xv * xv, axis=1, keepdims=True)

        f32 = jnp.float32
        L = (
            lax.broadcasted_iota(jnp.int32, (m, LANES), 0) % LANES
            == lax.broadcasted_iota(jnp.int32, (m, LANES), 1)
        ).astype(f32)
        Rt = (
            lax.broadcasted_iota(jnp.int32, (nblk, m), 1) // LANES
            == lax.broadcasted_iota(jnp.int32, (nblk, m), 0)
        ).astype(f32)

        packed_ref[:, :] = jnp.dot(
            Rt, L * ssq, preferred_element_type=f32
        )

        pl.semaphore_wait(barrier_sem, 1)

        rdma = pltpu.make_async_remote_copy(
            src_ref=packed_ref,
            dst_ref=recv_ref,
            send_sem=send_sem,
            recv_sem=recv_sem,
            device_id=nbr,
            device_id_type=pl.DeviceIdType.MESH,
        )
        rdma.start()
        xg = xv * g_ref[0, :]
        R = (
            lax.broadcasted_iota(jnp.int32, (m, nblk), 0) // LANES
            == lax.broadcasted_iota(jnp.int32, (m, nblk), 1)
        ).astype(f32)
        rdma.wait()

        total = packed_ref[:, :] + recv_ref[:, :]
        v = jnp.sum(
            jnp.dot(R, total, preferred_element_type=f32) * L,
            axis=1,
            keepdims=True,
        )
        inv_rms = lax.rsqrt(v * (1.0 / N_GLOBAL_COLS) + EPS)
        out_ref[:, :] = xg * inv_rms

    return pl.pallas_call(
        body,
        out_shape=jax.ShapeDtypeStruct((m, n), x.dtype),
        in_specs=[
            pl.BlockSpec(memory_space=pltpu.VMEM),
            pl.BlockSpec(memory_space=pltpu.VMEM),
        ],
        out_specs=pl.BlockSpec(memory_space=pltpu.VMEM),
        scratch_shapes=[
            pltpu.VMEM((nblk, LANES), jnp.float32),
            pltpu.VMEM((nblk, LANES), jnp.float32),
            pltpu.SemaphoreType.DMA,
            pltpu.SemaphoreType.DMA,
        ],
        compiler_params=pltpu.CompilerParams(collective_id=0),
    )(x, gamma2d)
